# baseline (speedup 1.0000x reference)
import functools

import jax
import jax.numpy as jnp
from jax import lax
from jax.experimental import pallas as pl
from jax.experimental.pallas import tpu as pltpu
from jax.experimental.pallas import tpu_sc as plsc

EMBED_DIM = 64
PAIR = 2 * EMBED_DIM


@functools.lru_cache(maxsize=None)
def _gather_fn(batch: int, num_rows: int):
    info = plsc.get_sparse_core_info()
    nw = info.num_cores * info.num_subcores
    rows_per_w = batch // nw
    assert batch % nw == 0

    mesh = plsc.VectorSubcoreMesh(core_axis_name="c", subcore_axis_name="s")

    @functools.partial(
        pl.kernel,
        mesh=mesh,
        out_type=jax.ShapeDtypeStruct((batch, PAIR), jnp.float32),
        scratch_types=[
            pltpu.VMEM((rows_per_w, PAIR), jnp.float32),
            pltpu.VMEM_SHARED((batch // 2,), jnp.int32),
            pltpu.SMEM((rows_per_w,), jnp.int32),
            pltpu.SemaphoreType.DMA,
        ],
    )
    def gather(table, ids, out, rows_v, idx_sh, idx_s, sem):
        wid = lax.axis_index("s") * info.num_cores + lax.axis_index("c")
        base = wid * rows_per_w
        sub = lax.axis_index("s") * rows_per_w
        pltpu.sync_copy(ids.at[pl.ds(base, rows_per_w)],
                        idx_sh.at[pl.ds(sub, rows_per_w)])
        pltpu.sync_copy(idx_sh.at[pl.ds(sub, rows_per_w)], idx_s)

        def fire(j, _):
            pltpu.make_async_copy(
                table.at[pl.ds(idx_s[j], 1)],
                rows_v.at[pl.ds(j, 1)],
                sem,
            ).start()
            return _

        lax.fori_loop(0, rows_per_w, fire, 0)

        def drain(j, _):
            pltpu.make_async_copy(
                table.at[pl.ds(0, 1)],
                rows_v.at[pl.ds(0, 1)],
                sem,
            ).wait()
            return _

        lax.fori_loop(0, rows_per_w, drain, 0)
        pltpu.sync_copy(rows_v, out.at[pl.ds(base, rows_per_w)])

    return gather


BLOCK_C = 2048
HALF_C = BLOCK_C // 2


def _relayout_body(t_ref, out_ref):
    y = jnp.transpose(t_ref[:])
    out_ref[:] = jnp.concatenate([y[:HALF_C], y[HALF_C:]], axis=1)


def _relayout(tT):
    v = tT.shape[1]
    nblocks = (v + BLOCK_C - 1) // BLOCK_C
    return pl.pallas_call(
        _relayout_body,
        grid=(nblocks,),
        in_specs=[pl.BlockSpec((EMBED_DIM, BLOCK_C), lambda c: (0, c))],
        out_specs=pl.BlockSpec((HALF_C, PAIR), lambda c: (c, 0)),
        out_shape=jax.ShapeDtypeStruct((HALF_C * nblocks, PAIR), jnp.float32),
    )(tT)


def _mlp_body(pu_ref, pm_ref, up_ref, mp_ref, w1u_ref, w1m_ref, b1_ref,
              w2_ref, b2_ref, w3_ref, b3_ref, out_ref):
    u = jnp.where(up_ref[:] > 0, pu_ref[:, EMBED_DIM:], pu_ref[:, :EMBED_DIM])
    m = jnp.where(mp_ref[:] > 0, pm_ref[:, EMBED_DIM:], pm_ref[:, :EMBED_DIM])
    h = u @ w1u_ref[:] + m @ w1m_ref[:] + b1_ref[:]
    h = jnp.maximum(h, 0.0)
    h = jnp.maximum(h @ w2_ref[:] + b2_ref[:], 0.0)
    out_ref[:] = jnp.sum(h * w3_ref[:], axis=1) + b3_ref[0, 0]


def _mlp(pu, pm, up, mp, w1u, w1m, b1r, w2, b2r, w3r, b3r, block_b: int,
         interpret=False):
    batch = pu.shape[0]
    h1 = w1u.shape[1]
    h2 = w2.shape[1]
    grid = (batch // block_b,)
    return pl.pallas_call(
        _mlp_body,
        grid=grid,
        in_specs=[
            pl.BlockSpec((block_b, PAIR), lambda i: (i, 0)),
            pl.BlockSpec((block_b, PAIR), lambda i: (i, 0)),
            pl.BlockSpec((block_b, 1), lambda i: (i, 0)),
            pl.BlockSpec((block_b, 1), lambda i: (i, 0)),
            pl.BlockSpec((EMBED_DIM, h1), lambda i: (0, 0)),
            pl.BlockSpec((EMBED_DIM, h1), lambda i: (0, 0)),
            pl.BlockSpec((1, h1), lambda i: (0, 0)),
            pl.BlockSpec((h1, h2), lambda i: (0, 0)),
            pl.BlockSpec((1, h2), lambda i: (0, 0)),
            pl.BlockSpec((1, h2), lambda i: (0, 0)),
            pl.BlockSpec((1, 1), lambda i: (0, 0)),
        ],
        out_specs=pl.BlockSpec((block_b,), lambda i: (i,)),
        out_shape=jax.ShapeDtypeStruct((batch,), jnp.float32),
        interpret=interpret,
    )(pu, pm, up, mp, w1u, w1m, b1r, w2, b2r, w3r, b3r)


def kernel(user_ids, movie_ids, user_table, movie_table, W1, b1, W2, b2, W3, b3):
    batch = user_ids.shape[0]
    uids = user_ids.astype(jnp.int32)
    mids = movie_ids.astype(jnp.int32)
    ut2 = _relayout(user_table.T)
    mt2 = _relayout(movie_table.T)
    gather_u = _gather_fn(batch, ut2.shape[0])
    gather_m = _gather_fn(batch, mt2.shape[0])
    upj = (uids // BLOCK_C) * HALF_C + (uids % HALF_C)
    mpj = (mids // BLOCK_C) * HALF_C + (mids % HALF_C)
    pu = gather_u(ut2, upj)
    pm = gather_m(mt2, mpj)
    up = ((uids % BLOCK_C) // HALF_C).astype(jnp.float32).reshape(batch, 1)
    mp = ((mids % BLOCK_C) // HALF_C).astype(jnp.float32).reshape(batch, 1)
    return _mlp(
        pu, pm, up, mp, W1[:EMBED_DIM], W1[EMBED_DIM:],
        b1.reshape(1, -1), W2, b2.reshape(1, -1),
        W3.reshape(1, -1), b3.reshape(1, 1),
        block_b=2048,
    )


# speedup vs baseline: 1.6685x; 1.6685x over previous
import functools

import jax
import jax.numpy as jnp
from jax import lax
from jax.experimental import pallas as pl
from jax.experimental.pallas import tpu as pltpu
from jax.experimental.pallas import tpu_sc as plsc

EMBED_DIM = 64
PAIR = 2 * EMBED_DIM


@functools.lru_cache(maxsize=None)
def _gather_fn(batch: int, num_rows: int):
    info = plsc.get_sparse_core_info()
    nw = info.num_cores * info.num_subcores
    rows_per_w = batch // nw
    assert batch % nw == 0

    mesh = plsc.VectorSubcoreMesh(core_axis_name="c", subcore_axis_name="s")

    @functools.partial(
        pl.kernel,
        mesh=mesh,
        out_type=jax.ShapeDtypeStruct((batch, PAIR), jnp.float32),
        scratch_types=[
            pltpu.VMEM((rows_per_w, PAIR), jnp.float32),
            pltpu.VMEM_SHARED((batch // 2,), jnp.int32),
            pltpu.SMEM((rows_per_w,), jnp.int32),
            pltpu.SemaphoreType.DMA,
        ],
    )
    def gather(table, ids, out, rows_v, idx_sh, idx_s, sem):
        wid = lax.axis_index("s") * info.num_cores + lax.axis_index("c")
        base = wid * rows_per_w
        sub = lax.axis_index("s") * rows_per_w
        pltpu.sync_copy(ids.at[pl.ds(base, rows_per_w)],
                        idx_sh.at[pl.ds(sub, rows_per_w)])
        pltpu.sync_copy(idx_sh.at[pl.ds(sub, rows_per_w)], idx_s)

        def fire(j, _):
            pltpu.make_async_copy(
                table.at[pl.ds(idx_s[j], 1)],
                rows_v.at[pl.ds(j, 1)],
                sem,
            ).start()
            return _

        lax.fori_loop(0, rows_per_w, fire, 0)

        def drain(j, _):
            pltpu.make_async_copy(
                table.at[pl.ds(0, 1)],
                rows_v.at[pl.ds(0, 1)],
                sem,
            ).wait()
            return _

        lax.fori_loop(0, rows_per_w, drain, 0)
        pltpu.sync_copy(rows_v, out.at[pl.ds(base, rows_per_w)])

    return gather


BLOCK_C = 8192
HALF_C = BLOCK_C // 2


def _relayout_body(t_ref, out_ref):
    y = jnp.transpose(t_ref[:])
    out_ref[:] = jnp.concatenate([y[:HALF_C], y[HALF_C:]], axis=1)


def _relayout(tT):
    v = tT.shape[1]
    nblocks = (v + BLOCK_C - 1) // BLOCK_C
    return pl.pallas_call(
        _relayout_body,
        grid=(nblocks,),
        in_specs=[pl.BlockSpec((EMBED_DIM, BLOCK_C), lambda c: (0, c))],
        out_specs=pl.BlockSpec((HALF_C, PAIR), lambda c: (c, 0)),
        out_shape=jax.ShapeDtypeStruct((HALF_C * nblocks, PAIR), jnp.float32),
    )(tT)


def _mlp_body(pu_ref, pm_ref, up_ref, mp_ref, w1u_ref, w1m_ref, b1_ref,
              w2_ref, b2_ref, w3_ref, b3_ref, out_ref):
    u = jnp.where(up_ref[:] > 0, pu_ref[:, EMBED_DIM:], pu_ref[:, :EMBED_DIM])
    m = jnp.where(mp_ref[:] > 0, pm_ref[:, EMBED_DIM:], pm_ref[:, :EMBED_DIM])
    h = u @ w1u_ref[:] + m @ w1m_ref[:] + b1_ref[:]
    h = jnp.maximum(h, 0.0)
    h = jnp.maximum(h @ w2_ref[:] + b2_ref[:], 0.0)
    out_ref[:] = jnp.sum(h * w3_ref[:], axis=1) + b3_ref[0, 0]


def _mlp(pu, pm, up, mp, w1u, w1m, b1r, w2, b2r, w3r, b3r, block_b: int,
         interpret=False):
    batch = pu.shape[0]
    h1 = w1u.shape[1]
    h2 = w2.shape[1]
    grid = (batch // block_b,)
    return pl.pallas_call(
        _mlp_body,
        grid=grid,
        in_specs=[
            pl.BlockSpec((block_b, PAIR), lambda i: (i, 0)),
            pl.BlockSpec((block_b, PAIR), lambda i: (i, 0)),
            pl.BlockSpec((block_b, 1), lambda i: (i, 0)),
            pl.BlockSpec((block_b, 1), lambda i: (i, 0)),
            pl.BlockSpec((EMBED_DIM, h1), lambda i: (0, 0)),
            pl.BlockSpec((EMBED_DIM, h1), lambda i: (0, 0)),
            pl.BlockSpec((1, h1), lambda i: (0, 0)),
            pl.BlockSpec((h1, h2), lambda i: (0, 0)),
            pl.BlockSpec((1, h2), lambda i: (0, 0)),
            pl.BlockSpec((1, h2), lambda i: (0, 0)),
            pl.BlockSpec((1, 1), lambda i: (0, 0)),
        ],
        out_specs=pl.BlockSpec((block_b,), lambda i: (i,)),
        out_shape=jax.ShapeDtypeStruct((batch,), jnp.float32),
        interpret=interpret,
    )(pu, pm, up, mp, w1u, w1m, b1r, w2, b2r, w3r, b3r)


def kernel(user_ids, movie_ids, user_table, movie_table, W1, b1, W2, b2, W3, b3):
    batch = user_ids.shape[0]
    uids = user_ids.astype(jnp.int32)
    mids = movie_ids.astype(jnp.int32)
    ut2 = _relayout(user_table.T)
    mt2 = _relayout(movie_table.T)
    gather_u = _gather_fn(batch, ut2.shape[0])
    gather_m = _gather_fn(batch, mt2.shape[0])
    upj = (uids // BLOCK_C) * HALF_C + (uids % HALF_C)
    mpj = (mids // BLOCK_C) * HALF_C + (mids % HALF_C)
    pu = gather_u(ut2, upj)
    pm = gather_m(mt2, mpj)
    up = ((uids % BLOCK_C) // HALF_C).astype(jnp.float32).reshape(batch, 1)
    mp = ((mids % BLOCK_C) // HALF_C).astype(jnp.float32).reshape(batch, 1)
    return _mlp(
        pu, pm, up, mp, W1[:EMBED_DIM], W1[EMBED_DIM:],
        b1.reshape(1, -1), W2, b2.reshape(1, -1),
        W3.reshape(1, -1), b3.reshape(1, 1),
        block_b=2048,
    )


# speedup vs baseline: 1.8801x; 1.1268x over previous
import functools

import jax
import jax.numpy as jnp
from jax import lax
from jax.experimental import pallas as pl
from jax.experimental.pallas import tpu as pltpu
from jax.experimental.pallas import tpu_sc as plsc

EMBED_DIM = 64
PAIR = 2 * EMBED_DIM


@functools.lru_cache(maxsize=None)
def _gather_fn(batch: int, num_rows: int):
    info = plsc.get_sparse_core_info()
    nw = info.num_cores * info.num_subcores
    rows_per_w = batch // nw
    assert batch % nw == 0

    mesh = plsc.VectorSubcoreMesh(core_axis_name="c", subcore_axis_name="s")

    @functools.partial(
        pl.kernel,
        mesh=mesh,
        out_type=jax.ShapeDtypeStruct((batch, PAIR), jnp.float32),
        scratch_types=[
            pltpu.VMEM((rows_per_w, PAIR), jnp.float32),
            pltpu.VMEM_SHARED((batch // 2,), jnp.int32),
            pltpu.SMEM((rows_per_w,), jnp.int32),
            pltpu.SemaphoreType.DMA,
        ],
    )
    def gather(table, ids, out, rows_v, idx_sh, idx_s, sem):
        wid = lax.axis_index("s") * info.num_cores + lax.axis_index("c")
        base = wid * rows_per_w
        sub = lax.axis_index("s") * rows_per_w
        pltpu.sync_copy(ids.at[pl.ds(base, rows_per_w)],
                        idx_sh.at[pl.ds(sub, rows_per_w)])
        pltpu.sync_copy(idx_sh.at[pl.ds(sub, rows_per_w)], idx_s)

        def fire(j, _):
            pltpu.make_async_copy(
                table.at[pl.ds(idx_s[j], 1)],
                rows_v.at[pl.ds(j, 1)],
                sem,
            ).start()
            return _

        lax.fori_loop(0, rows_per_w, fire, 0)

        def drain(j, _):
            pltpu.make_async_copy(
                table.at[pl.ds(0, 1)],
                rows_v.at[pl.ds(0, 1)],
                sem,
            ).wait()
            return _

        lax.fori_loop(0, rows_per_w, drain, 0)
        pltpu.sync_copy(rows_v, out.at[pl.ds(base, rows_per_w)])

    return gather


BLOCK_C = 16384
HALF_C = BLOCK_C // 2


def _relayout_body(t_ref, out_ref):
    y = jnp.transpose(t_ref[:])
    out_ref[:] = jnp.concatenate([y[:HALF_C], y[HALF_C:]], axis=1)


def _relayout(tT):
    v = tT.shape[1]
    nblocks = (v + BLOCK_C - 1) // BLOCK_C
    return pl.pallas_call(
        _relayout_body,
        grid=(nblocks,),
        in_specs=[pl.BlockSpec((EMBED_DIM, BLOCK_C), lambda c: (0, c))],
        out_specs=pl.BlockSpec((HALF_C, PAIR), lambda c: (c, 0)),
        out_shape=jax.ShapeDtypeStruct((HALF_C * nblocks, PAIR), jnp.float32),
    )(tT)


def _mlp_body(pu_ref, pm_ref, up_ref, mp_ref, w1u_ref, w1m_ref, b1_ref,
              w2_ref, b2_ref, w3_ref, b3_ref, out_ref):
    u = jnp.where(up_ref[:] > 0, pu_ref[:, EMBED_DIM:], pu_ref[:, :EMBED_DIM])
    m = jnp.where(mp_ref[:] > 0, pm_ref[:, EMBED_DIM:], pm_ref[:, :EMBED_DIM])
    h = u @ w1u_ref[:] + m @ w1m_ref[:] + b1_ref[:]
    h = jnp.maximum(h, 0.0)
    h = jnp.maximum(h @ w2_ref[:] + b2_ref[:], 0.0)
    out_ref[:] = jnp.sum(h * w3_ref[:], axis=1) + b3_ref[0, 0]


def _mlp(pu, pm, up, mp, w1u, w1m, b1r, w2, b2r, w3r, b3r, block_b: int,
         interpret=False):
    batch = pu.shape[0]
    h1 = w1u.shape[1]
    h2 = w2.shape[1]
    grid = (batch // block_b,)
    return pl.pallas_call(
        _mlp_body,
        grid=grid,
        in_specs=[
            pl.BlockSpec((block_b, PAIR), lambda i: (i, 0)),
            pl.BlockSpec((block_b, PAIR), lambda i: (i, 0)),
            pl.BlockSpec((block_b, 1), lambda i: (i, 0)),
            pl.BlockSpec((block_b, 1), lambda i: (i, 0)),
            pl.BlockSpec((EMBED_DIM, h1), lambda i: (0, 0)),
            pl.BlockSpec((EMBED_DIM, h1), lambda i: (0, 0)),
            pl.BlockSpec((1, h1), lambda i: (0, 0)),
            pl.BlockSpec((h1, h2), lambda i: (0, 0)),
            pl.BlockSpec((1, h2), lambda i: (0, 0)),
            pl.BlockSpec((1, h2), lambda i: (0, 0)),
            pl.BlockSpec((1, 1), lambda i: (0, 0)),
        ],
        out_specs=pl.BlockSpec((block_b,), lambda i: (i,)),
        out_shape=jax.ShapeDtypeStruct((batch,), jnp.float32),
        interpret=interpret,
    )(pu, pm, up, mp, w1u, w1m, b1r, w2, b2r, w3r, b3r)


def kernel(user_ids, movie_ids, user_table, movie_table, W1, b1, W2, b2, W3, b3):
    batch = user_ids.shape[0]
    uids = user_ids.astype(jnp.int32)
    mids = movie_ids.astype(jnp.int32)
    ut2 = _relayout(user_table.T)
    mt2 = _relayout(movie_table.T)
    gather_u = _gather_fn(batch, ut2.shape[0])
    gather_m = _gather_fn(batch, mt2.shape[0])
    upj = (uids // BLOCK_C) * HALF_C + (uids % HALF_C)
    mpj = (mids // BLOCK_C) * HALF_C + (mids % HALF_C)
    pu = gather_u(ut2, upj)
    pm = gather_m(mt2, mpj)
    up = ((uids % BLOCK_C) // HALF_C).astype(jnp.float32).reshape(batch, 1)
    mp = ((mids % BLOCK_C) // HALF_C).astype(jnp.float32).reshape(batch, 1)
    return _mlp(
        pu, pm, up, mp, W1[:EMBED_DIM], W1[EMBED_DIM:],
        b1.reshape(1, -1), W2, b2.reshape(1, -1),
        W3.reshape(1, -1), b3.reshape(1, 1),
        block_b=2048,
    )


# speedup vs baseline: 1.9952x; 1.0612x over previous
import functools

import jax
import jax.numpy as jnp
from jax import lax
from jax.experimental import pallas as pl
from jax.experimental.pallas import tpu as pltpu
from jax.experimental.pallas import tpu_sc as plsc

EMBED_DIM = 64
PAIR = 2 * EMBED_DIM


@functools.lru_cache(maxsize=None)
def _gather_fn(batch: int, num_rows: int):
    info = plsc.get_sparse_core_info()
    nw = info.num_cores * info.num_subcores
    rows_per_w = batch // nw
    assert batch % nw == 0

    mesh = plsc.VectorSubcoreMesh(core_axis_name="c", subcore_axis_name="s")

    @functools.partial(
        pl.kernel,
        mesh=mesh,
        out_type=jax.ShapeDtypeStruct((batch, PAIR), jnp.float32),
        scratch_types=[
            pltpu.VMEM((rows_per_w, PAIR), jnp.float32),
            pltpu.VMEM_SHARED((batch // 2,), jnp.int32),
            pltpu.SMEM((rows_per_w,), jnp.int32),
            pltpu.SemaphoreType.DMA,
        ],
    )
    def gather(table, ids, out, rows_v, idx_sh, idx_s, sem):
        wid = lax.axis_index("s") * info.num_cores + lax.axis_index("c")
        base = wid * rows_per_w
        sub = lax.axis_index("s") * rows_per_w
        pltpu.sync_copy(ids.at[pl.ds(base, rows_per_w)],
                        idx_sh.at[pl.ds(sub, rows_per_w)])
        pltpu.sync_copy(idx_sh.at[pl.ds(sub, rows_per_w)], idx_s)

        def fire(j, _):
            pltpu.make_async_copy(
                table.at[pl.ds(idx_s[j], 1)],
                rows_v.at[pl.ds(j, 1)],
                sem,
            ).start()
            return _

        lax.fori_loop(0, rows_per_w, fire, 0)

        def drain(j, _):
            pltpu.make_async_copy(
                table.at[pl.ds(0, 1)],
                rows_v.at[pl.ds(0, 1)],
                sem,
            ).wait()
            return _

        lax.fori_loop(0, rows_per_w, drain, 0)
        pltpu.sync_copy(rows_v, out.at[pl.ds(base, rows_per_w)])

    return gather


BLOCK_C = 32768
HALF_C = BLOCK_C // 2


def _relayout_body(t_ref, out_ref):
    y = jnp.transpose(t_ref[:])
    out_ref[:] = jnp.concatenate([y[:HALF_C], y[HALF_C:]], axis=1)


def _relayout(tT):
    v = tT.shape[1]
    nblocks = (v + BLOCK_C - 1) // BLOCK_C
    return pl.pallas_call(
        _relayout_body,
        grid=(nblocks,),
        in_specs=[pl.BlockSpec((EMBED_DIM, BLOCK_C), lambda c: (0, c))],
        out_specs=pl.BlockSpec((HALF_C, PAIR), lambda c: (c, 0)),
        out_shape=jax.ShapeDtypeStruct((HALF_C * nblocks, PAIR), jnp.float32),
    )(tT)


def _mlp_body(pu_ref, pm_ref, up_ref, mp_ref, w1u_ref, w1m_ref, b1_ref,
              w2_ref, b2_ref, w3_ref, b3_ref, out_ref):
    u = jnp.where(up_ref[:] > 0, pu_ref[:, EMBED_DIM:], pu_ref[:, :EMBED_DIM])
    m = jnp.where(mp_ref[:] > 0, pm_ref[:, EMBED_DIM:], pm_ref[:, :EMBED_DIM])
    h = u @ w1u_ref[:] + m @ w1m_ref[:] + b1_ref[:]
    h = jnp.maximum(h, 0.0)
    h = jnp.maximum(h @ w2_ref[:] + b2_ref[:], 0.0)
    out_ref[:] = jnp.sum(h * w3_ref[:], axis=1) + b3_ref[0, 0]


def _mlp(pu, pm, up, mp, w1u, w1m, b1r, w2, b2r, w3r, b3r, block_b: int,
         interpret=False):
    batch = pu.shape[0]
    h1 = w1u.shape[1]
    h2 = w2.shape[1]
    grid = (batch // block_b,)
    return pl.pallas_call(
        _mlp_body,
        grid=grid,
        in_specs=[
            pl.BlockSpec((block_b, PAIR), lambda i: (i, 0)),
            pl.BlockSpec((block_b, PAIR), lambda i: (i, 0)),
            pl.BlockSpec((block_b, 1), lambda i: (i, 0)),
            pl.BlockSpec((block_b, 1), lambda i: (i, 0)),
            pl.BlockSpec((EMBED_DIM, h1), lambda i: (0, 0)),
            pl.BlockSpec((EMBED_DIM, h1), lambda i: (0, 0)),
            pl.BlockSpec((1, h1), lambda i: (0, 0)),
            pl.BlockSpec((h1, h2), lambda i: (0, 0)),
            pl.BlockSpec((1, h2), lambda i: (0, 0)),
            pl.BlockSpec((1, h2), lambda i: (0, 0)),
            pl.BlockSpec((1, 1), lambda i: (0, 0)),
        ],
        out_specs=pl.BlockSpec((block_b,), lambda i: (i,)),
        out_shape=jax.ShapeDtypeStruct((batch,), jnp.float32),
        interpret=interpret,
    )(pu, pm, up, mp, w1u, w1m, b1r, w2, b2r, w3r, b3r)


def kernel(user_ids, movie_ids, user_table, movie_table, W1, b1, W2, b2, W3, b3):
    batch = user_ids.shape[0]
    uids = user_ids.astype(jnp.int32)
    mids = movie_ids.astype(jnp.int32)
    ut2 = _relayout(user_table.T)
    mt2 = _relayout(movie_table.T)
    gather_u = _gather_fn(batch, ut2.shape[0])
    gather_m = _gather_fn(batch, mt2.shape[0])
    upj = (uids // BLOCK_C) * HALF_C + (uids % HALF_C)
    mpj = (mids // BLOCK_C) * HALF_C + (mids % HALF_C)
    pu = gather_u(ut2, upj)
    pm = gather_m(mt2, mpj)
    up = ((uids % BLOCK_C) // HALF_C).astype(jnp.float32).reshape(batch, 1)
    mp = ((mids % BLOCK_C) // HALF_C).astype(jnp.float32).reshape(batch, 1)
    return _mlp(
        pu, pm, up, mp, W1[:EMBED_DIM], W1[EMBED_DIM:],
        b1.reshape(1, -1), W2, b2.reshape(1, -1),
        W3.reshape(1, -1), b3.reshape(1, 1),
        block_b=2048,
    )


# speedup vs baseline: 2.2251x; 1.1152x over previous
import functools

import jax
import jax.numpy as jnp
from jax import lax
from jax.experimental import pallas as pl
from jax.experimental.pallas import tpu as pltpu
from jax.experimental.pallas import tpu_sc as plsc

EMBED_DIM = 64
PAIR = 2 * EMBED_DIM


@functools.lru_cache(maxsize=None)
def _gather_fn(batch: int, num_rows: int):
    info = plsc.get_sparse_core_info()
    nw = info.num_cores * info.num_subcores
    rows_per_w = batch // nw
    assert batch % nw == 0

    mesh = plsc.VectorSubcoreMesh(core_axis_name="c", subcore_axis_name="s")

    @functools.partial(
        pl.kernel,
        mesh=mesh,
        out_type=jax.ShapeDtypeStruct((batch, EMBED_DIM), jnp.int32),
        scratch_types=[
            pltpu.VMEM((rows_per_w, EMBED_DIM), jnp.int32),
            pltpu.VMEM_SHARED((batch // 2,), jnp.int32),
            pltpu.SMEM((rows_per_w,), jnp.int32),
            pltpu.SemaphoreType.DMA,
        ],
    )
    def gather(table, ids, out, rows_v, idx_sh, idx_s, sem):
        wid = lax.axis_index("s") * info.num_cores + lax.axis_index("c")
        base = wid * rows_per_w
        sub = lax.axis_index("s") * rows_per_w
        pltpu.sync_copy(ids.at[pl.ds(base, rows_per_w)],
                        idx_sh.at[pl.ds(sub, rows_per_w)])
        pltpu.sync_copy(idx_sh.at[pl.ds(sub, rows_per_w)], idx_s)

        def fire(j, _):
            pltpu.make_async_copy(
                table.at[pl.ds(idx_s[j], 1)],
                rows_v.at[pl.ds(j, 1)],
                sem,
            ).start()
            return _

        lax.fori_loop(0, rows_per_w, fire, 0)

        def drain(j, _):
            pltpu.make_async_copy(
                table.at[pl.ds(0, 1)],
                rows_v.at[pl.ds(0, 1)],
                sem,
            ).wait()
            return _

        lax.fori_loop(0, rows_per_w, drain, 0)
        pltpu.sync_copy(rows_v, out.at[pl.ds(base, rows_per_w)])

    return gather


BLOCK_C = 32768
HALF_C = BLOCK_C // 2


def _relayout_body(t_ref, out_ref):
    y = jnp.transpose(t_ref[:])
    bits = lax.bitcast_convert_type(y, jnp.int32)
    r = bits + jnp.int32(0x7FFF) + ((bits >> 16) & jnp.int32(1))
    lo = (r[:HALF_C] >> 16) & jnp.int32(0xFFFF)
    hi = r[HALF_C:] & jnp.int32(-65536)
    out_ref[:] = lo | hi


def _relayout(tT):
    v = tT.shape[1]
    nblocks = (v + BLOCK_C - 1) // BLOCK_C
    return pl.pallas_call(
        _relayout_body,
        grid=(nblocks,),
        in_specs=[pl.BlockSpec((EMBED_DIM, BLOCK_C), lambda c: (0, c))],
        out_specs=pl.BlockSpec((HALF_C, EMBED_DIM), lambda c: (c, 0)),
        out_shape=jax.ShapeDtypeStruct((HALF_C * nblocks, EMBED_DIM),
                                       jnp.int32),
    )(tT)


def _mlp_body(pu_ref, pm_ref, up_ref, mp_ref, w1u_ref, w1m_ref, b1_ref,
              w2_ref, b2_ref, w3_ref, b3_ref, out_ref):
    u = jnp.where(up_ref[:] > 0, pu_ref[:, EMBED_DIM:],
                  pu_ref[:, :EMBED_DIM]).astype(jnp.float32)
    m = jnp.where(mp_ref[:] > 0, pm_ref[:, EMBED_DIM:],
                  pm_ref[:, :EMBED_DIM]).astype(jnp.float32)
    h = u @ w1u_ref[:] + m @ w1m_ref[:] + b1_ref[:]
    h = jnp.maximum(h, 0.0)
    h = jnp.maximum(h @ w2_ref[:] + b2_ref[:], 0.0)
    out_ref[:] = jnp.sum(h * w3_ref[:], axis=1) + b3_ref[0, 0]


def _mlp(pu, pm, up, mp, w1u, w1m, b1r, w2, b2r, w3r, b3r, block_b: int,
         interpret=False):
    batch = pu.shape[0]
    h1 = w1u.shape[1]
    h2 = w2.shape[1]
    grid = (batch // block_b,)
    return pl.pallas_call(
        _mlp_body,
        grid=grid,
        in_specs=[
            pl.BlockSpec((block_b, PAIR), lambda i: (i, 0)),
            pl.BlockSpec((block_b, PAIR), lambda i: (i, 0)),
            pl.BlockSpec((block_b, 1), lambda i: (i, 0)),
            pl.BlockSpec((block_b, 1), lambda i: (i, 0)),
            pl.BlockSpec((EMBED_DIM, h1), lambda i: (0, 0)),
            pl.BlockSpec((EMBED_DIM, h1), lambda i: (0, 0)),
            pl.BlockSpec((1, h1), lambda i: (0, 0)),
            pl.BlockSpec((h1, h2), lambda i: (0, 0)),
            pl.BlockSpec((1, h2), lambda i: (0, 0)),
            pl.BlockSpec((1, h2), lambda i: (0, 0)),
            pl.BlockSpec((1, 1), lambda i: (0, 0)),
        ],
        out_specs=pl.BlockSpec((block_b,), lambda i: (i,)),
        out_shape=jax.ShapeDtypeStruct((batch,), jnp.float32),
        interpret=interpret,
    )(pu, pm, up, mp, w1u, w1m, b1r, w2, b2r, w3r, b3r)


def _unpack(p_i32):
    pr = lax.bitcast_convert_type(p_i32, jnp.bfloat16)
    return jnp.concatenate([pr[:, :, 0], pr[:, :, 1]], axis=-1)


def kernel(user_ids, movie_ids, user_table, movie_table, W1, b1, W2, b2, W3, b3):
    batch = user_ids.shape[0]
    uids = user_ids.astype(jnp.int32)
    mids = movie_ids.astype(jnp.int32)
    ut2 = _relayout(user_table.T)
    mt2 = _relayout(movie_table.T)
    gather_u = _gather_fn(batch, ut2.shape[0])
    gather_m = _gather_fn(batch, mt2.shape[0])
    upj = (uids // BLOCK_C) * HALF_C + (uids % HALF_C)
    mpj = (mids // BLOCK_C) * HALF_C + (mids % HALF_C)
    pu = _unpack(gather_u(ut2, upj))
    pm = _unpack(gather_m(mt2, mpj))
    up = ((uids % BLOCK_C) // HALF_C).astype(jnp.float32).reshape(batch, 1)
    mp = ((mids % BLOCK_C) // HALF_C).astype(jnp.float32).reshape(batch, 1)
    return _mlp(
        pu, pm, up, mp, W1[:EMBED_DIM], W1[EMBED_DIM:],
        b1.reshape(1, -1), W2, b2.reshape(1, -1),
        W3.reshape(1, -1), b3.reshape(1, 1),
        block_b=2048,
    )
